# Initial kernel scaffold; baseline (speedup 1.0000x reference)
import jax
import jax.numpy as jnp
from jax.experimental import pallas as pl


def kernel(x, edge_index, w1, a1, w2, a2, bn_gamma, bn_beta):
    raise NotImplementedError("write your pallas kernel here")


# speedup vs baseline: 1.4325x; 1.4325x over previous
import functools

import jax
import jax.numpy as jnp
from jax import lax
from jax.experimental import pallas as pl
from jax.experimental.pallas import tpu as pltpu
from jax.experimental.pallas import tpu_sc as plsc

N = 10000
E = 160000
F = 256
HID = 256
NH1 = 8
NEG_SLOPE = 0.2
EPS_BN = 1e-5

NP = 10240
EP = 163840
PAD_NODE = NP - 1

NC = 2
NS = 16
NW = NC * NS
EPW = EP // NW
BLK = 128
NBLK = EPW // BLK
ZROWS = NP // NS

RM = 256


def _elu(v):
    return jnp.where(v > 0, v, jnp.exp(jnp.minimum(v, 0.0)) - 1.0)


def _mm_call(x, w, ncols):
    k = x.shape[1]

    def body(x_ref, w_ref, *outs):
        xb = x_ref[...]
        wb = w_ref[...]
        for c in range(ncols):
            outs[c][...] = jnp.dot(xb, wb[:, c * 128:(c + 1) * 128],
                                   preferred_element_type=jnp.float32)

    return pl.pallas_call(
        body,
        grid=(NP // RM,),
        in_specs=[pl.BlockSpec((RM, k), lambda i: (i, 0)),
                  pl.BlockSpec((k, ncols * 128), lambda i: (0, 0))],
        out_specs=[pl.BlockSpec((RM, 128), lambda i: (i, 0))] * ncols,
        out_shape=[jax.ShapeDtypeStruct((NP, 128), jnp.float32)] * ncols,
    )(x, w)


def _alpha_call(h_chunks, amat):
    ncols = len(h_chunks)

    def body(amat_ref, *refs):
        hs = [refs[c][...] for c in range(ncols)]
        hcat = jnp.concatenate(hs, axis=1)
        refs[ncols][...] = jnp.dot(hcat, amat_ref[...],
                                   preferred_element_type=jnp.float32)

    return pl.pallas_call(
        body,
        grid=(NP // RM,),
        in_specs=[pl.BlockSpec(amat.shape, lambda i: (0, 0))] +
                 [pl.BlockSpec((RM, 128), lambda i: (i, 0))] * ncols,
        out_specs=pl.BlockSpec((RM, 32), lambda i: (i, 0)),
        out_shape=jax.ShapeDtypeStruct((NP, 32), jnp.float32),
    )(amat, *h_chunks)


def _combine1_call(rs, hp_chunks):
    nch = len(hp_chunks)

    def body(rs_ref, *refs):
        r = rs_ref[...]
        rsum = r[0] + r[1]
        cols = []
        for c in range(nch):
            hp = refs[c][...]
            s = hp[0] + hp[1]
            denom = rsum[:, c // 2][:, None] + 1e-16
            cols.append(_elu(s / denom))
        refs[nch][...] = jnp.concatenate(cols, axis=1)

    return pl.pallas_call(
        body,
        grid=(NP // RM,),
        in_specs=[pl.BlockSpec((2, RM, 16), lambda i: (0, i, 0))] +
                 [pl.BlockSpec((2, RM, 128), lambda i: (0, i, 0))] * nch,
        out_specs=pl.BlockSpec((RM, nch * 128), lambda i: (i, 0)),
        out_shape=jax.ShapeDtypeStruct((NP, nch * 128), jnp.float32),
    )(rs, *hp_chunks)


def _final_call(rs, hp_chunks, gamma, beta):
    nch = len(hp_chunks)
    inv = float(1.0 / (1.0 + EPS_BN) ** 0.5)

    def body(rs_ref, g_ref, b_ref, *refs):
        r = rs_ref[...]
        denom = (r[0] + r[1])[:, 0:1] + 1e-16
        parts = []
        for c in range(nch):
            hp = refs[c][...]
            parts.append(hp[0] + hp[1])
        s = jnp.concatenate(parts, axis=1)
        x2 = s / denom
        xn = x2 * (g_ref[...] * inv) + b_ref[...]
        refs[nch][...] = _elu(xn)

    return pl.pallas_call(
        body,
        grid=(NP // RM,),
        in_specs=[pl.BlockSpec((2, RM, 16), lambda i: (0, i, 0)),
                  pl.BlockSpec((1, nch * 128), lambda i: (0, 0)),
                  pl.BlockSpec((1, nch * 128), lambda i: (0, 0))] +
                 [pl.BlockSpec((2, RM, 128), lambda i: (0, i, 0))] * nch,
        out_specs=pl.BlockSpec((RM, nch * 128), lambda i: (i, 0)),
        out_shape=jax.ShapeDtypeStruct((NP, nch * 128), jnp.float32),
    )(rs, gamma.reshape(1, -1), beta.reshape(1, -1), *hp_chunks)


def _mesh():
    return plsc.VectorSubcoreMesh(core_axis_name="c", subcore_axis_name="s")


def _edgew_call(acat, srcv, dstv, nh):
    @functools.partial(
        pl.kernel,
        mesh=_mesh(),
        compiler_params=pltpu.CompilerParams(use_tc_tiling_on_sc=False),
        out_type=[jax.ShapeDtypeStruct((EP, 16), jnp.float32),
                  jax.ShapeDtypeStruct((NC, NP, 16), jnp.float32)],
        scratch_types=[
            pltpu.VMEM((BLK,), jnp.int32),
            pltpu.VMEM((BLK,), jnp.int32),
            pltpu.VMEM((BLK, 32), jnp.float32),
            pltpu.VMEM((BLK, 32), jnp.float32),
            pltpu.VMEM((BLK, 16), jnp.float32),
            pltpu.VMEM_SHARED((NP, 16), jnp.float32),
            pltpu.SemaphoreType.DMA,
            pltpu.SemaphoreType.DMA,
        ],
    )
    def k(acat_h, src_h, dst_h, wt_out, rs_out,
          sidx, didx, abuf, dbuf, wrow, rsacc, sem1, sem2):
        cid = lax.axis_index("c")
        tid = lax.axis_index("s")
        wid = tid * NC + cid

        def zero_wrow(i, carry):
            wrow[i, :] = jnp.zeros((16,), jnp.float32)
            return carry

        lax.fori_loop(0, BLK, zero_wrow, 0)
        for piece in range(ZROWS // BLK):
            pltpu.sync_copy(wrow, rsacc.at[pl.ds(tid * ZROWS + piece * BLK, BLK)])
        plsc.subcore_barrier()

        ebase = wid * EPW

        def block(b, carry):
            base = ebase + b * BLK
            pltpu.sync_copy(src_h.at[pl.ds(base, BLK)], sidx)
            pltpu.sync_copy(dst_h.at[pl.ds(base, BLK)], didx)
            cp1 = pltpu.async_copy(acat_h.at[sidx], abuf, sem1)
            cp2 = pltpu.async_copy(acat_h.at[didx], dbuf, sem2)
            cp1.wait()
            cp2.wait()

            def edge(e, c2):
                s = abuf[e, pl.ds(0, 16)]
                d = dbuf[e, pl.ds(16, 16)]
                ev = s + d
                lr = jnp.where(ev > 0, ev, NEG_SLOPE * ev)
                w = jnp.exp(-lr)
                wrow[e, :] = w
                return c2

            lax.fori_loop(0, BLK, edge, 0)
            pltpu.sync_copy(wrow, rsacc.at[sidx], add=True)
            pltpu.sync_copy(wrow, wt_out.at[pl.ds(base, BLK)])
            return carry

        lax.fori_loop(0, NBLK, block, 0)
        plsc.subcore_barrier()
        for piece in range(ZROWS // BLK):
            off = tid * ZROWS + piece * BLK
            pltpu.sync_copy(rsacc.at[pl.ds(off, BLK)],
                            rs_out.at[cid, pl.ds(off, BLK)])

    return k(acat, srcv, dstv)


def _featpass_call(hc, wt, srcv, dstv, head):
    @functools.partial(
        pl.kernel,
        mesh=_mesh(),
        compiler_params=pltpu.CompilerParams(use_tc_tiling_on_sc=False),
        out_type=jax.ShapeDtypeStruct((NC, NP, 128), jnp.float32),
        scratch_types=[
            pltpu.VMEM((BLK,), jnp.int32),
            pltpu.VMEM((BLK,), jnp.int32),
            pltpu.VMEM((BLK, 16), jnp.float32),
            pltpu.VMEM((BLK, 128), jnp.float32),
            pltpu.VMEM_SHARED((NP, 128), jnp.float32),
            pltpu.SemaphoreType.DMA,
        ],
    )
    def k(hc_h, wt_h, src_h, dst_h, out_h,
          sidx, didx, wbuf, rows, acc, sem):
        cid = lax.axis_index("c")
        tid = lax.axis_index("s")
        wid = tid * NC + cid

        def zero_rows(i, carry):
            for j in range(8):
                rows[i, pl.ds(j * 16, 16)] = jnp.zeros((16,), jnp.float32)
            return carry

        lax.fori_loop(0, BLK, zero_rows, 0)
        for piece in range(ZROWS // BLK):
            pltpu.sync_copy(rows, acc.at[pl.ds(tid * ZROWS + piece * BLK, BLK)])
        plsc.subcore_barrier()

        ebase = wid * EPW

        def block(b, carry):
            base = ebase + b * BLK
            pltpu.sync_copy(src_h.at[pl.ds(base, BLK)], sidx)
            pltpu.sync_copy(dst_h.at[pl.ds(base, BLK)], didx)
            pltpu.sync_copy(wt_h.at[pl.ds(base, BLK)], wbuf)
            pltpu.async_copy(hc_h.at[didx], rows, sem).wait()

            def edge(e, c2):
                wrow = wbuf[e, :]
                wv = jnp.zeros((16,), jnp.float32) + wrow[head]
                for j in range(8):
                    rows[e, pl.ds(j * 16, 16)] = rows[e, pl.ds(j * 16, 16)] * wv
                return c2

            lax.fori_loop(0, BLK, edge, 0)
            pltpu.sync_copy(rows, acc.at[sidx], add=True)
            return carry

        lax.fori_loop(0, NBLK, block, 0)
        plsc.subcore_barrier()
        for piece in range(ZROWS // BLK):
            off = tid * ZROWS + piece * BLK
            pltpu.sync_copy(acc.at[pl.ds(off, BLK)],
                            out_h.at[cid, pl.ds(off, BLK)])

    return k(hc, wt, srcv, dstv)


def kernel(x, edge_index, w1, a1, w2, a2, bn_gamma, bn_beta):
    f32 = jnp.float32
    x_pad = jnp.pad(x, ((0, NP - N), (0, 0)))
    pad_idx = jnp.full((EP - E,), PAD_NODE, jnp.int32)
    srcp = jnp.concatenate([edge_index[0], pad_idx])
    dstp = jnp.concatenate([edge_index[1], pad_idx])

    w_all = jnp.transpose(w1, (1, 0, 2)).reshape(F, NH1 * HID)
    amat1 = jnp.zeros((NH1 * HID, 32), f32)
    for h in range(NH1):
        amat1 = amat1.at[h * HID:(h + 1) * HID, h].set(a1[h, :HID])
        amat1 = amat1.at[h * HID:(h + 1) * HID, 16 + h].set(a1[h, HID:])
    amat2 = (jnp.zeros((HID, 32), f32)
             .at[:, 0].set(a2[:HID])
             .at[:, 16].set(a2[HID:]))

    h1 = _mm_call(x_pad, w_all, ncols=16)
    acat1 = _alpha_call(h1, amat1)
    wt1, rs1 = _edgew_call(acat1, srcp, dstp, nh=NH1)
    hp1 = [_featpass_call(h1[c], wt1, srcp, dstp, head=c // 2)
           for c in range(16)]
    x1 = _combine1_call(rs1, hp1)

    h2 = _mm_call(x1, w2, ncols=2)
    acat2 = _alpha_call(h2, amat2)
    wt2, rs2 = _edgew_call(acat2, srcp, dstp, nh=1)
    hp2 = [_featpass_call(h2[c], wt2, srcp, dstp, head=0)
           for c in range(2)]
    out = _final_call(rs2, hp2, bn_gamma, bn_beta)
    return out[:N]


# speedup vs baseline: 1.9627x; 1.3702x over previous
import functools

import jax
import jax.numpy as jnp
from jax import lax
from jax.experimental import pallas as pl
from jax.experimental.pallas import tpu as pltpu
from jax.experimental.pallas import tpu_sc as plsc

N = 10000
E = 160000
F = 256
HID = 256
NH1 = 8
NEG_SLOPE = 0.2
EPS_BN = 1e-5

NP = 10240
EP = 163840
PAD_NODE = NP - 1

NC = 2
NS = 16
NW = NC * NS
EPW = EP // NW
BLK = 128
NBLK = EPW // BLK
ZROWS = NP // NS

RM = 256


def _elu(v):
    return jnp.where(v > 0, v, jnp.exp(jnp.minimum(v, 0.0)) - 1.0)


def _mm_call(x, w, ncols):
    k = x.shape[1]

    def body(x_ref, w_ref, o_ref):
        o_ref[0] = jnp.dot(x_ref[...], w_ref[...],
                           preferred_element_type=jnp.float32)

    return pl.pallas_call(
        body,
        grid=(NP // RM, ncols),
        in_specs=[pl.BlockSpec((RM, k), lambda i, c: (i, 0)),
                  pl.BlockSpec((k, 128), lambda i, c: (0, c))],
        out_specs=pl.BlockSpec((1, RM, 128), lambda i, c: (c, i, 0)),
        out_shape=jax.ShapeDtypeStruct((ncols, NP, 128), jnp.float32),
    )(x, w)


def _alpha_call(h3, amat):
    nch = h3.shape[0]

    def body(amat_ref, h_ref, o_ref):
        hb = h_ref[...]
        hcat = jnp.concatenate([hb[c] for c in range(nch)], axis=1)
        o_ref[...] = jnp.dot(hcat, amat_ref[...],
                             preferred_element_type=jnp.float32)

    return pl.pallas_call(
        body,
        grid=(NP // RM,),
        in_specs=[pl.BlockSpec(amat.shape, lambda i: (0, 0)),
                  pl.BlockSpec((nch, RM, 128), lambda i: (0, i, 0))],
        out_specs=pl.BlockSpec((RM, 32), lambda i: (i, 0)),
        out_shape=jax.ShapeDtypeStruct((NP, 32), jnp.float32),
    )(amat, h3)


def _combine1_call(rs, hp):
    nch = hp.shape[0]

    def body(rs_ref, hp_ref, o_ref):
        c = pl.program_id(1)
        r = rs_ref[...]
        rsum = r[0] + r[1]
        onehot = (lax.broadcasted_iota(jnp.int32, (1, 16), 1) == c // 2)
        denom = jnp.sum(jnp.where(onehot, rsum, 0.0), axis=1,
                        keepdims=True) + 1e-16
        h = hp_ref[...]
        s = h[0, 0] + h[0, 1]
        o_ref[...] = _elu(s / denom)

    return pl.pallas_call(
        body,
        grid=(NP // RM, nch),
        in_specs=[pl.BlockSpec((2, RM, 16), lambda i, c: (0, i, 0)),
                  pl.BlockSpec((1, 2, RM, 128), lambda i, c: (c, 0, i, 0))],
        out_specs=pl.BlockSpec((RM, 128), lambda i, c: (i, c)),
        out_shape=jax.ShapeDtypeStruct((NP, nch * 128), jnp.float32),
    )(rs, hp)


def _final_call(rs, hp, gamma, beta):
    nch = hp.shape[0]
    inv = float(1.0 / (1.0 + EPS_BN) ** 0.5)

    def body(rs_ref, g_ref, b_ref, hp_ref, o_ref):
        r = rs_ref[...]
        denom = (r[0] + r[1])[:, 0:1] + 1e-16
        h = hp_ref[...]
        s = jnp.concatenate([h[c, 0] + h[c, 1] for c in range(nch)], axis=1)
        x2 = s / denom
        xn = x2 * (g_ref[...] * inv) + b_ref[...]
        o_ref[...] = _elu(xn)

    return pl.pallas_call(
        body,
        grid=(NP // RM,),
        in_specs=[pl.BlockSpec((2, RM, 16), lambda i: (0, i, 0)),
                  pl.BlockSpec((1, nch * 128), lambda i: (0, 0)),
                  pl.BlockSpec((1, nch * 128), lambda i: (0, 0)),
                  pl.BlockSpec((nch, 2, RM, 128), lambda i: (0, 0, i, 0))],
        out_specs=pl.BlockSpec((RM, nch * 128), lambda i: (i, 0)),
        out_shape=jax.ShapeDtypeStruct((NP, nch * 128), jnp.float32),
    )(rs, gamma.reshape(1, -1), beta.reshape(1, -1), hp)


def _mesh():
    return plsc.VectorSubcoreMesh(core_axis_name="c", subcore_axis_name="s")


def _edgew_call(acat, src2d, dst2d, nh):
    @functools.partial(
        pl.kernel,
        mesh=_mesh(),
        compiler_params=pltpu.CompilerParams(use_tc_tiling_on_sc=False),
        out_type=[jax.ShapeDtypeStruct((EP, 16), jnp.float32),
                  jax.ShapeDtypeStruct((NC, NP, 16), jnp.float32)],
        scratch_types=[
            pltpu.VMEM((NBLK, BLK), jnp.int32),
            pltpu.VMEM((NBLK, BLK), jnp.int32),
            pltpu.VMEM((BLK, 32), jnp.float32),
            pltpu.VMEM((BLK, 32), jnp.float32),
            pltpu.VMEM((BLK, 32), jnp.float32),
            pltpu.VMEM((BLK, 32), jnp.float32),
            pltpu.VMEM((BLK, 16), jnp.float32),
            pltpu.VMEM_SHARED((NP, 16), jnp.float32),
            pltpu.SemaphoreType.DMA,
            pltpu.SemaphoreType.DMA,
            pltpu.SemaphoreType.DMA,
            pltpu.SemaphoreType.DMA,
        ],
    )
    def k(acat_h, src_h, dst_h, wt_out, rs_out,
          s2d, d2d, as0, ad0, as1, ad1, wrow, rsacc, sa0, sb0, sa1, sb1):
        cid = lax.axis_index("c")
        tid = lax.axis_index("s")
        wid = tid * NC + cid

        def zero_wrow(i, carry):
            wrow[i, :] = jnp.zeros((16,), jnp.float32)
            return carry

        lax.fori_loop(0, BLK, zero_wrow, 0)
        for piece in range(ZROWS // BLK):
            pltpu.sync_copy(wrow, rsacc.at[pl.ds(tid * ZROWS + piece * BLK, BLK)])
        plsc.subcore_barrier()

        brow = wid * NBLK
        ebase = wid * EPW
        pltpu.sync_copy(src_h.at[pl.ds(brow, NBLK)], s2d)
        pltpu.sync_copy(dst_h.at[pl.ds(brow, NBLK)], d2d)

        def issue(b, abuf, dbuf, sema, semb):
            pltpu.async_copy(acat_h.at[s2d.at[b]], abuf, sema)
            pltpu.async_copy(acat_h.at[d2d.at[b]], dbuf, semb)

        def wait(b, abuf, dbuf, sema, semb):
            pltpu.make_async_copy(acat_h.at[s2d.at[b]], abuf, sema).wait()
            pltpu.make_async_copy(acat_h.at[d2d.at[b]], dbuf, semb).wait()

        def phase(b, abuf, dbuf, sema, semb, anx, dnx, semna, semnb):
            wait(b, abuf, dbuf, sema, semb)

            @pl.when(b + 1 < NBLK)
            def _():
                issue(b + 1, anx, dnx, semna, semnb)

            def edge(e, c2):
                s = abuf[e, pl.ds(0, 16)]
                d = dbuf[e, pl.ds(16, 16)]
                ev = s + d
                lr = jnp.where(ev > 0, ev, NEG_SLOPE * ev)
                wrow[e, :] = jnp.exp(-lr)
                return c2

            lax.fori_loop(0, BLK, edge, 0)
            pltpu.sync_copy(wrow, rsacc.at[s2d.at[b]], add=True)
            pltpu.sync_copy(wrow, wt_out.at[pl.ds(ebase + b * BLK, BLK)])

        issue(0, as0, ad0, sa0, sb0)

        def pair(g, carry):
            phase(2 * g, as0, ad0, sa0, sb0, as1, ad1, sa1, sb1)
            phase(2 * g + 1, as1, ad1, sa1, sb1, as0, ad0, sa0, sb0)
            return carry

        lax.fori_loop(0, NBLK // 2, pair, 0)
        plsc.subcore_barrier()
        for piece in range(ZROWS // BLK):
            off = tid * ZROWS + piece * BLK
            pltpu.sync_copy(rsacc.at[pl.ds(off, BLK)],
                            rs_out.at[cid, pl.ds(off, BLK)])

    return k(acat, src2d, dst2d)


def _featpass_call(h3, wt, src2d, dst2d, nh):
    ch = h3.shape[0]
    cph = ch // nh

    @functools.partial(
        pl.kernel,
        mesh=_mesh(),
        compiler_params=pltpu.CompilerParams(use_tc_tiling_on_sc=False),
        out_type=jax.ShapeDtypeStruct((ch, NC, NP, 128), jnp.float32),
        scratch_types=[
            pltpu.VMEM((NBLK, BLK), jnp.int32),
            pltpu.VMEM((NBLK, BLK), jnp.int32),
            pltpu.VMEM((BLK, 16), jnp.float32),
            pltpu.VMEM((BLK, 16), jnp.float32),
            pltpu.VMEM((BLK, 128), jnp.float32),
            pltpu.VMEM((BLK, 128), jnp.float32),
            pltpu.VMEM_SHARED((NP, 128), jnp.float32),
            pltpu.SemaphoreType.DMA,
            pltpu.SemaphoreType.DMA,
            pltpu.SemaphoreType.DMA,
            pltpu.SemaphoreType.DMA,
        ],
    )
    def k(h3_h, wt_h, src_h, dst_h, out_h,
          s2d, d2d, wb0, wb1, r0, r1, acc, sg0, sg1, sw0, sw1):
        cid = lax.axis_index("c")
        tid = lax.axis_index("s")
        wid = tid * NC + cid

        def zero_r0(i, carry):
            for j in range(8):
                r0[i, pl.ds(j * 16, 16)] = jnp.zeros((16,), jnp.float32)
            return carry

        lax.fori_loop(0, BLK, zero_r0, 0)

        brow = wid * NBLK
        ebase = wid * EPW
        pltpu.sync_copy(src_h.at[pl.ds(brow, NBLK)], s2d)
        pltpu.sync_copy(dst_h.at[pl.ds(brow, NBLK)], d2d)

        for piece in range(ZROWS // BLK):
            pltpu.sync_copy(r0, acc.at[pl.ds(tid * ZROWS + piece * BLK, BLK)])
        plsc.subcore_barrier()

        def chunk(c, carry):
            chead = c // cph
            lanes_c = jnp.zeros((16,), jnp.int32) + chead
            hc = h3_h.at[c]

            def issue(b, rbuf, wbuf, semg, semw):
                pltpu.async_copy(hc.at[d2d.at[b]], rbuf, semg)
                pltpu.async_copy(wt_h.at[pl.ds(ebase + b * BLK, BLK)],
                                 wbuf, semw)

            def wait(b, rbuf, wbuf, semg, semw):
                pltpu.make_async_copy(hc.at[d2d.at[b]], rbuf, semg).wait()
                pltpu.make_async_copy(wt_h.at[pl.ds(ebase + b * BLK, BLK)],
                                      wbuf, semw).wait()

            def phase(b, rbuf, wbuf, semg, semw, rnx, wnx, semgn, semwn):
                wait(b, rbuf, wbuf, semg, semw)

                @pl.when(b + 1 < NBLK)
                def _():
                    issue(b + 1, rnx, wnx, semgn, semwn)

                def edge(e, c2):
                    wrow = wbuf[e, :]
                    wv = wrow[lanes_c]
                    for j in range(8):
                        rbuf[e, pl.ds(j * 16, 16)] = (
                            rbuf[e, pl.ds(j * 16, 16)] * wv)
                    return c2

                lax.fori_loop(0, BLK, edge, 0)
                pltpu.sync_copy(rbuf, acc.at[s2d.at[b]], add=True)

            issue(0, r0, wb0, sg0, sw0)

            def pair(g, c2):
                phase(2 * g, r0, wb0, sg0, sw0, r1, wb1, sg1, sw1)
                phase(2 * g + 1, r1, wb1, sg1, sw1, r0, wb0, sg0, sw0)
                return c2

            lax.fori_loop(0, NBLK // 2, pair, 0)
            plsc.subcore_barrier()
            lax.fori_loop(0, BLK, zero_r0, 0)
            for piece in range(ZROWS // BLK):
                off = tid * ZROWS + piece * BLK
                pltpu.sync_copy(acc.at[pl.ds(off, BLK)],
                                out_h.at[c, cid, pl.ds(off, BLK)])
                pltpu.sync_copy(r0, acc.at[pl.ds(off, BLK)])
            plsc.subcore_barrier()
            return carry

        lax.fori_loop(0, ch, chunk, 0)

    return k(h3, wt, src2d, dst2d)


def kernel(x, edge_index, w1, a1, w2, a2, bn_gamma, bn_beta):
    f32 = jnp.float32
    x_pad = jnp.pad(x, ((0, NP - N), (0, 0)))
    pad_idx = jnp.full((EP - E,), PAD_NODE, jnp.int32)
    src2d = jnp.concatenate([edge_index[0], pad_idx]).reshape(EP // BLK, BLK)
    dst2d = jnp.concatenate([edge_index[1], pad_idx]).reshape(EP // BLK, BLK)

    w_all = jnp.transpose(w1, (1, 0, 2)).reshape(F, NH1 * HID)
    amat1 = jnp.zeros((NH1 * HID, 32), f32)
    for h in range(NH1):
        amat1 = amat1.at[h * HID:(h + 1) * HID, h].set(a1[h, :HID])
        amat1 = amat1.at[h * HID:(h + 1) * HID, 16 + h].set(a1[h, HID:])
    amat2 = (jnp.zeros((HID, 32), f32)
             .at[:, 0].set(a2[:HID])
             .at[:, 16].set(a2[HID:]))

    h1 = _mm_call(x_pad, w_all, ncols=16)
    acat1 = _alpha_call(h1, amat1)
    wt1, rs1 = _edgew_call(acat1, src2d, dst2d, nh=NH1)
    hp1 = _featpass_call(h1, wt1, src2d, dst2d, nh=NH1)
    x1 = _combine1_call(rs1, hp1)

    h2 = _mm_call(x1, w2, ncols=2)
    acat2 = _alpha_call(h2, amat2)
    wt2, rs2 = _edgew_call(acat2, src2d, dst2d, nh=1)
    hp2 = _featpass_call(h2, wt2, src2d, dst2d, nh=1)
    out = _final_call(rs2, hp2, bn_gamma, bn_beta)
    return out[:N]


# speedup vs baseline: 2.3999x; 1.2227x over previous
import functools

import jax
import jax.numpy as jnp
from jax import lax
from jax.experimental import pallas as pl
from jax.experimental.pallas import tpu as pltpu
from jax.experimental.pallas import tpu_sc as plsc

N = 10000
E = 160000
F = 256
HID = 256
NH1 = 8
NEG_SLOPE = 0.2
EPS_BN = 1e-5

NP = 10240
EP = 163840
PAD_NODE = NP - 1

NC = 2
NS = 16
NW = NC * NS
EPW = EP // NW
BLK = 128
NBLK = EPW // BLK
ZROWS = NP // NS
FBLK = 80
FEPW = EP // NS
FNBLK = FEPW // FBLK

RM = 256


def _elu(v):
    return jnp.where(v > 0, v, jnp.exp(jnp.minimum(v, 0.0)) - 1.0)


def _mm_call(x, w, ncols):
    k = x.shape[1]

    def body(x_ref, w_ref, o_ref):
        o_ref[0] = jnp.dot(x_ref[...], w_ref[...],
                           preferred_element_type=jnp.float32)

    return pl.pallas_call(
        body,
        grid=(NP // RM, ncols),
        in_specs=[pl.BlockSpec((RM, k), lambda i, c: (i, 0)),
                  pl.BlockSpec((k, 128), lambda i, c: (0, c))],
        out_specs=pl.BlockSpec((1, RM, 128), lambda i, c: (c, i, 0)),
        out_shape=jax.ShapeDtypeStruct((ncols, NP, 128), jnp.float32),
    )(x, w)


def _alpha_call(h3, amat):
    nch = h3.shape[0]

    def body(amat_ref, h_ref, o_ref):
        hb = h_ref[...]
        hcat = jnp.concatenate([hb[c] for c in range(nch)], axis=1)
        o_ref[...] = jnp.dot(hcat, amat_ref[...],
                             preferred_element_type=jnp.float32)

    return pl.pallas_call(
        body,
        grid=(NP // RM,),
        in_specs=[pl.BlockSpec(amat.shape, lambda i: (0, 0)),
                  pl.BlockSpec((nch, RM, 128), lambda i: (0, i, 0))],
        out_specs=pl.BlockSpec((RM, 32), lambda i: (i, 0)),
        out_shape=jax.ShapeDtypeStruct((NP, 32), jnp.float32),
    )(amat, h3)


def _combine1_call(rs, hp):
    nch = hp.shape[0]

    def body(rs_ref, hp_ref, o_ref):
        c = pl.program_id(1)
        r = rs_ref[...]
        rsum = r[0] + r[1]
        onehot = (lax.broadcasted_iota(jnp.int32, (1, 16), 1) == c // 2)
        denom = jnp.sum(jnp.where(onehot, rsum, 0.0), axis=1,
                        keepdims=True) + 1e-16
        s = hp_ref[...][0]
        o_ref[...] = _elu(s / denom)

    return pl.pallas_call(
        body,
        grid=(NP // RM, nch),
        in_specs=[pl.BlockSpec((2, RM, 16), lambda i, c: (0, i, 0)),
                  pl.BlockSpec((1, RM, 128), lambda i, c: (c, i, 0))],
        out_specs=pl.BlockSpec((RM, 128), lambda i, c: (i, c)),
        out_shape=jax.ShapeDtypeStruct((NP, nch * 128), jnp.float32),
    )(rs, hp)


def _final_call(rs, hp, gamma, beta):
    nch = hp.shape[0]
    inv = float(1.0 / (1.0 + EPS_BN) ** 0.5)

    def body(rs_ref, g_ref, b_ref, hp_ref, o_ref):
        r = rs_ref[...]
        denom = (r[0] + r[1])[:, 0:1] + 1e-16
        h = hp_ref[...]
        s = jnp.concatenate([h[c] for c in range(nch)], axis=1)
        x2 = s / denom
        xn = x2 * (g_ref[...] * inv) + b_ref[...]
        o_ref[...] = _elu(xn)

    return pl.pallas_call(
        body,
        grid=(NP // RM,),
        in_specs=[pl.BlockSpec((2, RM, 16), lambda i: (0, i, 0)),
                  pl.BlockSpec((1, nch * 128), lambda i: (0, 0)),
                  pl.BlockSpec((1, nch * 128), lambda i: (0, 0)),
                  pl.BlockSpec((nch, RM, 128), lambda i: (0, i, 0))],
        out_specs=pl.BlockSpec((RM, nch * 128), lambda i: (i, 0)),
        out_shape=jax.ShapeDtypeStruct((NP, nch * 128), jnp.float32),
    )(rs, gamma.reshape(1, -1), beta.reshape(1, -1), hp)


def _mesh():
    return plsc.VectorSubcoreMesh(core_axis_name="c", subcore_axis_name="s")


def _edgew_call(acat, src2d, dst2d, nh):
    @functools.partial(
        pl.kernel,
        mesh=_mesh(),
        compiler_params=pltpu.CompilerParams(use_tc_tiling_on_sc=False),
        out_type=[jax.ShapeDtypeStruct((EP, 16), jnp.float32),
                  jax.ShapeDtypeStruct((NC, NP, 16), jnp.float32)],
        scratch_types=[
            pltpu.VMEM((NBLK, BLK), jnp.int32),
            pltpu.VMEM((NBLK, BLK), jnp.int32),
            pltpu.VMEM((BLK, 32), jnp.float32),
            pltpu.VMEM((BLK, 32), jnp.float32),
            pltpu.VMEM((BLK, 32), jnp.float32),
            pltpu.VMEM((BLK, 32), jnp.float32),
            pltpu.VMEM((BLK, 16), jnp.float32),
            pltpu.VMEM_SHARED((NP, 16), jnp.float32),
            pltpu.SemaphoreType.DMA,
            pltpu.SemaphoreType.DMA,
            pltpu.SemaphoreType.DMA,
            pltpu.SemaphoreType.DMA,
        ],
    )
    def k(acat_h, src_h, dst_h, wt_out, rs_out,
          s2d, d2d, as0, ad0, as1, ad1, wrow, rsacc, sa0, sb0, sa1, sb1):
        cid = lax.axis_index("c")
        tid = lax.axis_index("s")
        wid = tid * NC + cid

        def zero_wrow(i, carry):
            wrow[i, :] = jnp.zeros((16,), jnp.float32)
            return carry

        lax.fori_loop(0, BLK, zero_wrow, 0)
        for piece in range(ZROWS // BLK):
            pltpu.sync_copy(wrow, rsacc.at[pl.ds(tid * ZROWS + piece * BLK, BLK)])
        plsc.subcore_barrier()

        brow = wid * NBLK
        ebase = wid * EPW
        pltpu.sync_copy(src_h.at[pl.ds(brow, NBLK)], s2d)
        pltpu.sync_copy(dst_h.at[pl.ds(brow, NBLK)], d2d)

        def issue(b, abuf, dbuf, sema, semb):
            pltpu.async_copy(acat_h.at[s2d.at[b]], abuf, sema)
            pltpu.async_copy(acat_h.at[d2d.at[b]], dbuf, semb)

        def wait(b, abuf, dbuf, sema, semb):
            pltpu.make_async_copy(acat_h.at[s2d.at[b]], abuf, sema).wait()
            pltpu.make_async_copy(acat_h.at[d2d.at[b]], dbuf, semb).wait()

        def phase(b, abuf, dbuf, sema, semb, anx, dnx, semna, semnb):
            wait(b, abuf, dbuf, sema, semb)

            @pl.when(b + 1 < NBLK)
            def _():
                issue(b + 1, anx, dnx, semna, semnb)

            def edge(e, c2):
                s = abuf[e, pl.ds(0, 16)]
                d = dbuf[e, pl.ds(16, 16)]
                ev = s + d
                lr = jnp.where(ev > 0, ev, NEG_SLOPE * ev)
                wrow[e, :] = jnp.exp(-lr)
                return c2

            lax.fori_loop(0, BLK, edge, 0)
            pltpu.sync_copy(wrow, rsacc.at[s2d.at[b]], add=True)
            pltpu.sync_copy(wrow, wt_out.at[pl.ds(ebase + b * BLK, BLK)])

        issue(0, as0, ad0, sa0, sb0)

        def pair(g, carry):
            phase(2 * g, as0, ad0, sa0, sb0, as1, ad1, sa1, sb1)
            phase(2 * g + 1, as1, ad1, sa1, sb1, as0, ad0, sa0, sb0)
            return carry

        lax.fori_loop(0, NBLK // 2, pair, 0)
        plsc.subcore_barrier()
        for piece in range(ZROWS // BLK):
            off = tid * ZROWS + piece * BLK
            pltpu.sync_copy(rsacc.at[pl.ds(off, BLK)],
                            rs_out.at[cid, pl.ds(off, BLK)])

    return k(acat, src2d, dst2d)


def _featpass_call(h3, wt, src2d, dst2d, nh):
    ch = h3.shape[0]
    cph = ch // nh
    chc = ch // NC

    @functools.partial(
        pl.kernel,
        mesh=_mesh(),
        compiler_params=pltpu.CompilerParams(use_tc_tiling_on_sc=False),
        out_type=jax.ShapeDtypeStruct((ch, NP, 128), jnp.float32),
        scratch_types=[
            pltpu.VMEM((FNBLK, FBLK), jnp.int32),
            pltpu.VMEM((FNBLK, FBLK), jnp.int32),
            pltpu.VMEM((FBLK, 16), jnp.float32),
            pltpu.VMEM((FBLK, 16), jnp.float32),
            pltpu.VMEM((FBLK, 128), jnp.float32),
            pltpu.VMEM((FBLK, 128), jnp.float32),
            pltpu.VMEM_SHARED((NP, 128), jnp.float32),
            pltpu.SemaphoreType.DMA,
            pltpu.SemaphoreType.DMA,
            pltpu.SemaphoreType.DMA,
            pltpu.SemaphoreType.DMA,
            pltpu.SemaphoreType.DMA,
            pltpu.SemaphoreType.DMA,
        ],
    )
    def k(h3_h, wt_h, src_h, dst_h, out_h,
          s2d, d2d, wb0, wb1, r0, r1, acc, sg0, sg1, sw0, sw1, ss0, ss1):
        cid = lax.axis_index("c")
        tid = lax.axis_index("s")

        def zero_r0(i, carry):
            for j in range(8):
                r0[i, pl.ds(j * 16, 16)] = jnp.zeros((16,), jnp.float32)
            return carry

        lax.fori_loop(0, FBLK, zero_r0, 0)

        brow = tid * FNBLK
        ebase = tid * FEPW
        pltpu.sync_copy(src_h.at[pl.ds(brow, FNBLK)], s2d)
        pltpu.sync_copy(dst_h.at[pl.ds(brow, FNBLK)], d2d)

        for piece in range(ZROWS // FBLK):
            pltpu.sync_copy(r0, acc.at[pl.ds(tid * ZROWS + piece * FBLK, FBLK)])
        plsc.subcore_barrier()

        def chunk(cl, carry):
            c = cid * chc + cl
            chead = c // cph
            lanes_c = jnp.zeros((16,), jnp.int32) + chead
            hc = h3_h.at[c]

            def issue(b, rbuf, wbuf, semg, semw):
                pltpu.async_copy(hc.at[d2d.at[b]], rbuf, semg)
                pltpu.async_copy(wt_h.at[pl.ds(ebase + b * FBLK, FBLK)],
                                 wbuf, semw)

            def wait(b, rbuf, wbuf, semg, semw):
                pltpu.make_async_copy(hc.at[d2d.at[b]], rbuf, semg).wait()
                pltpu.make_async_copy(wt_h.at[pl.ds(ebase + b * FBLK, FBLK)],
                                      wbuf, semw).wait()

            def wait_scatter(b, rbuf, sems):
                pltpu.make_async_copy(rbuf, acc.at[s2d.at[b]], sems).wait()

            def phase(b, rbuf, wbuf, semg, semw, sems,
                      rnx, wnx, semgn, semwn, semsn):
                wait(b, rbuf, wbuf, semg, semw)

                @plsc.parallel_loop(0, FBLK, unroll=4)
                def _(e):
                    wrow = wbuf[e, :]
                    wv = wrow[lanes_c]
                    for j in range(8):
                        rbuf[e, pl.ds(j * 16, 16)] = (
                            rbuf[e, pl.ds(j * 16, 16)] * wv)

                @pl.when(b >= 1)
                def _():
                    wait_scatter(b - 1, rnx, semsn)

                @pl.when(b + 1 < FNBLK)
                def _():
                    issue(b + 1, rnx, wnx, semgn, semwn)

                pltpu.async_copy(rbuf, acc.at[s2d.at[b]], sems, add=True)

            issue(0, r0, wb0, sg0, sw0)

            def pair(g, c2):
                phase(2 * g, r0, wb0, sg0, sw0, ss0, r1, wb1, sg1, sw1, ss1)
                phase(2 * g + 1, r1, wb1, sg1, sw1, ss1, r0, wb0, sg0, sw0, ss0)
                return c2

            lax.fori_loop(0, FNBLK // 2, pair, 0)
            wait_scatter(FNBLK - 1, r1, ss1)
            plsc.subcore_barrier()
            lax.fori_loop(0, FBLK, zero_r0, 0)
            for piece in range(ZROWS // FBLK):
                off = tid * ZROWS + piece * FBLK
                pltpu.sync_copy(acc.at[pl.ds(off, FBLK)],
                                out_h.at[c, pl.ds(off, FBLK)])
                pltpu.sync_copy(r0, acc.at[pl.ds(off, FBLK)])
            plsc.subcore_barrier()
            return carry

        lax.fori_loop(0, chc, chunk, 0)

    return k(h3, wt, src2d, dst2d)


def kernel(x, edge_index, w1, a1, w2, a2, bn_gamma, bn_beta):
    f32 = jnp.float32
    x_pad = jnp.pad(x, ((0, NP - N), (0, 0)))
    pad_idx = jnp.full((EP - E,), PAD_NODE, jnp.int32)
    srcp = jnp.concatenate([edge_index[0], pad_idx])
    dstp = jnp.concatenate([edge_index[1], pad_idx])
    src2d = srcp.reshape(EP // BLK, BLK)
    dst2d = dstp.reshape(EP // BLK, BLK)
    src2f = srcp.reshape(EP // FBLK, FBLK)
    dst2f = dstp.reshape(EP // FBLK, FBLK)

    w_all = jnp.transpose(w1, (1, 0, 2)).reshape(F, NH1 * HID)
    amat1 = jnp.zeros((NH1 * HID, 32), f32)
    for h in range(NH1):
        amat1 = amat1.at[h * HID:(h + 1) * HID, h].set(a1[h, :HID])
        amat1 = amat1.at[h * HID:(h + 1) * HID, 16 + h].set(a1[h, HID:])
    amat2 = (jnp.zeros((HID, 32), f32)
             .at[:, 0].set(a2[:HID])
             .at[:, 16].set(a2[HID:]))

    h1 = _mm_call(x_pad, w_all, ncols=16)
    acat1 = _alpha_call(h1, amat1)
    wt1, rs1 = _edgew_call(acat1, src2d, dst2d, nh=NH1)
    hp1 = _featpass_call(h1, wt1, src2f, dst2f, nh=NH1)
    x1 = _combine1_call(rs1, hp1)

    h2 = _mm_call(x1, w2, ncols=2)
    acat2 = _alpha_call(h2, amat2)
    wt2, rs2 = _edgew_call(acat2, src2d, dst2d, nh=1)
    hp2 = _featpass_call(h2, wt2, src2f, dst2f, nh=1)
    out = _final_call(rs2, hp2, bn_gamma, bn_beta)
    return out[:N]


# speedup vs baseline: 2.8704x; 1.1961x over previous
import functools

import jax
import jax.numpy as jnp
from jax import lax
from jax.experimental import pallas as pl
from jax.experimental.pallas import tpu as pltpu
from jax.experimental.pallas import tpu_sc as plsc

N = 10000
E = 160000
F = 256
HID = 256
NH1 = 8
NEG_SLOPE = 0.2
EPS_BN = 1e-5

NP = 10240
EP = 163840
PAD_NODE = NP - 1

NC = 2
NS = 16
NW = NC * NS
EPW = EP // NW
BLK = 128
NBLK = EPW // BLK
ZROWS = NP // NS
FBLK = 80
FEPW = EP // NS
FNBLK = FEPW // FBLK

RM = 256


def _elu(v):
    return jnp.where(v > 0, v, jnp.exp(jnp.minimum(v, 0.0)) - 1.0)


def _mm_call(x, w, ncols):
    k = x.shape[1]

    def body(x_ref, w_ref, o_ref):
        o_ref[0] = jnp.dot(x_ref[...], w_ref[...],
                           preferred_element_type=jnp.float32)

    return pl.pallas_call(
        body,
        grid=(NP // RM, ncols),
        in_specs=[pl.BlockSpec((RM, k), lambda i, c: (i, 0)),
                  pl.BlockSpec((k, 128), lambda i, c: (0, c))],
        out_specs=pl.BlockSpec((1, RM, 128), lambda i, c: (c, i, 0)),
        out_shape=jax.ShapeDtypeStruct((ncols, NP, 128), jnp.float32),
    )(x, w)


def _alpha_call(h3, amat):
    nch = h3.shape[0]

    def body(amat_ref, h_ref, o_ref):
        hb = h_ref[...]
        hcat = jnp.concatenate([hb[c] for c in range(nch)], axis=1)
        o_ref[...] = jnp.dot(hcat, amat_ref[...],
                             preferred_element_type=jnp.float32)

    return pl.pallas_call(
        body,
        grid=(NP // RM,),
        in_specs=[pl.BlockSpec(amat.shape, lambda i: (0, 0)),
                  pl.BlockSpec((nch, RM, 128), lambda i: (0, i, 0))],
        out_specs=pl.BlockSpec((RM, 32), lambda i: (i, 0)),
        out_shape=jax.ShapeDtypeStruct((NP, 32), jnp.float32),
    )(amat, h3)


def _combine1_call(rs, hp):
    nch = hp.shape[0]

    def body(rs_ref, hp_ref, o_ref):
        r = rs_ref[...]
        rsum = r[0] + r[1]
        h = hp_ref[...]
        cols = []
        for c in range(nch):
            denom = rsum[:, c // 2][:, None] + 1e-16
            cols.append(_elu(h[c] / denom))
        o_ref[...] = jnp.concatenate(cols, axis=1)

    return pl.pallas_call(
        body,
        grid=(NP // RM,),
        in_specs=[pl.BlockSpec((2, RM, 16), lambda i: (0, i, 0)),
                  pl.BlockSpec((nch, RM, 128), lambda i: (0, i, 0))],
        out_specs=pl.BlockSpec((RM, nch * 128), lambda i: (i, 0)),
        out_shape=jax.ShapeDtypeStruct((NP, nch * 128), jnp.float32),
    )(rs, hp)


def _final_call(rs, hp, gamma, beta):
    nch = hp.shape[0]
    inv = float(1.0 / (1.0 + EPS_BN) ** 0.5)

    def body(rs_ref, g_ref, b_ref, hp_ref, o_ref):
        r = rs_ref[...]
        denom = (r[0] + r[1])[:, 0:1] + 1e-16
        h = hp_ref[...]
        s = jnp.concatenate([h[c] for c in range(nch)], axis=1)
        x2 = s / denom
        xn = x2 * (g_ref[...] * inv) + b_ref[...]
        o_ref[...] = _elu(xn)

    return pl.pallas_call(
        body,
        grid=(NP // RM,),
        in_specs=[pl.BlockSpec((2, RM, 16), lambda i: (0, i, 0)),
                  pl.BlockSpec((1, nch * 128), lambda i: (0, 0)),
                  pl.BlockSpec((1, nch * 128), lambda i: (0, 0)),
                  pl.BlockSpec((nch, RM, 128), lambda i: (0, i, 0))],
        out_specs=pl.BlockSpec((RM, nch * 128), lambda i: (i, 0)),
        out_shape=jax.ShapeDtypeStruct((NP, nch * 128), jnp.float32),
    )(rs, gamma.reshape(1, -1), beta.reshape(1, -1), hp)


def _mesh():
    return plsc.VectorSubcoreMesh(core_axis_name="c", subcore_axis_name="s")


def _edgew_call(acat, src2d, dst2d, nh):
    @functools.partial(
        pl.kernel,
        mesh=_mesh(),
        compiler_params=pltpu.CompilerParams(use_tc_tiling_on_sc=False),
        out_type=[jax.ShapeDtypeStruct((EP, 16), jnp.float32),
                  jax.ShapeDtypeStruct((NC, NP, 16), jnp.float32)],
        scratch_types=[
            pltpu.VMEM((NBLK, BLK), jnp.int32),
            pltpu.VMEM((NBLK, BLK), jnp.int32),
            pltpu.VMEM((BLK, 32), jnp.float32),
            pltpu.VMEM((BLK, 32), jnp.float32),
            pltpu.VMEM((BLK, 32), jnp.float32),
            pltpu.VMEM((BLK, 32), jnp.float32),
            pltpu.VMEM((BLK, 16), jnp.float32),
            pltpu.VMEM_SHARED((NP, 16), jnp.float32),
            pltpu.SemaphoreType.DMA,
            pltpu.SemaphoreType.DMA,
            pltpu.SemaphoreType.DMA,
            pltpu.SemaphoreType.DMA,
        ],
    )
    def k(acat_h, src_h, dst_h, wt_out, rs_out,
          s2d, d2d, as0, ad0, as1, ad1, wrow, rsacc, sa0, sb0, sa1, sb1):
        cid = lax.axis_index("c")
        tid = lax.axis_index("s")
        wid = tid * NC + cid

        def zero_wrow(i, carry):
            wrow[i, :] = jnp.zeros((16,), jnp.float32)
            return carry

        lax.fori_loop(0, BLK, zero_wrow, 0)
        for piece in range(ZROWS // BLK):
            pltpu.sync_copy(wrow, rsacc.at[pl.ds(tid * ZROWS + piece * BLK, BLK)])
        plsc.subcore_barrier()

        brow = wid * NBLK
        ebase = wid * EPW
        pltpu.sync_copy(src_h.at[pl.ds(brow, NBLK)], s2d)
        pltpu.sync_copy(dst_h.at[pl.ds(brow, NBLK)], d2d)

        def issue(b, abuf, dbuf, sema, semb):
            pltpu.async_copy(acat_h.at[s2d.at[b]], abuf, sema)
            pltpu.async_copy(acat_h.at[d2d.at[b]], dbuf, semb)

        def wait(b, abuf, dbuf, sema, semb):
            pltpu.make_async_copy(acat_h.at[s2d.at[b]], abuf, sema).wait()
            pltpu.make_async_copy(acat_h.at[d2d.at[b]], dbuf, semb).wait()

        def phase(b, abuf, dbuf, sema, semb, anx, dnx, semna, semnb):
            wait(b, abuf, dbuf, sema, semb)

            @pl.when(b + 1 < NBLK)
            def _():
                issue(b + 1, anx, dnx, semna, semnb)

            def edge(e, c2):
                s = abuf[e, pl.ds(0, 16)]
                d = dbuf[e, pl.ds(16, 16)]
                ev = s + d
                lr = jnp.where(ev > 0, ev, NEG_SLOPE * ev)
                wrow[e, :] = jnp.exp(-lr)
                return c2

            lax.fori_loop(0, BLK, edge, 0)
            pltpu.sync_copy(wrow, rsacc.at[s2d.at[b]], add=True)
            pltpu.sync_copy(wrow, wt_out.at[pl.ds(ebase + b * BLK, BLK)])

        issue(0, as0, ad0, sa0, sb0)

        def pair(g, carry):
            phase(2 * g, as0, ad0, sa0, sb0, as1, ad1, sa1, sb1)
            phase(2 * g + 1, as1, ad1, sa1, sb1, as0, ad0, sa0, sb0)
            return carry

        lax.fori_loop(0, NBLK // 2, pair, 0)
        plsc.subcore_barrier()
        for piece in range(ZROWS // BLK):
            off = tid * ZROWS + piece * BLK
            pltpu.sync_copy(rsacc.at[pl.ds(off, BLK)],
                            rs_out.at[cid, pl.ds(off, BLK)])

    return k(acat, src2d, dst2d)


def _featpass_call(h3, wt, src2d, dst2d, nh):
    ch = h3.shape[0]
    cph = ch // nh
    chc = ch // NC

    @functools.partial(
        pl.kernel,
        mesh=_mesh(),
        compiler_params=pltpu.CompilerParams(use_tc_tiling_on_sc=False),
        out_type=jax.ShapeDtypeStruct((ch, NP, 128), jnp.float32),
        scratch_types=[
            pltpu.VMEM((FNBLK, FBLK), jnp.int32),
            pltpu.VMEM((FNBLK, FBLK), jnp.int32),
            pltpu.VMEM((FBLK, 16), jnp.float32),
            pltpu.VMEM((FBLK, 16), jnp.float32),
            pltpu.VMEM((FBLK, 128), jnp.float32),
            pltpu.VMEM((FBLK, 128), jnp.float32),
            pltpu.VMEM_SHARED((NP, 128), jnp.float32),
            pltpu.SemaphoreType.DMA,
            pltpu.SemaphoreType.DMA,
            pltpu.SemaphoreType.DMA,
            pltpu.SemaphoreType.DMA,
            pltpu.SemaphoreType.DMA,
            pltpu.SemaphoreType.DMA,
        ],
    )
    def k(h3_h, wt_h, src_h, dst_h, out_h,
          s2d, d2d, wb0, wb1, r0, r1, acc, sg0, sg1, sw0, sw1, ss0, ss1):
        cid = lax.axis_index("c")
        tid = lax.axis_index("s")

        def zero_r0(i, carry):
            for j in range(8):
                r0[i, pl.ds(j * 16, 16)] = jnp.zeros((16,), jnp.float32)
            return carry

        lax.fori_loop(0, FBLK, zero_r0, 0)

        brow = tid * FNBLK
        ebase = tid * FEPW
        pltpu.sync_copy(src_h.at[pl.ds(brow, FNBLK)], s2d)
        pltpu.sync_copy(dst_h.at[pl.ds(brow, FNBLK)], d2d)

        for piece in range(ZROWS // FBLK):
            pltpu.sync_copy(r0, acc.at[pl.ds(tid * ZROWS + piece * FBLK, FBLK)])
        plsc.subcore_barrier()

        def chunk(cl, carry):
            c = cid * chc + cl
            chead = c // cph
            lanes_c = jnp.zeros((16,), jnp.int32) + chead
            hc = h3_h.at[c]

            def issue(b, rbuf, wbuf, semg, semw):
                pltpu.async_copy(hc.at[d2d.at[b]], rbuf, semg)
                pltpu.async_copy(wt_h.at[pl.ds(ebase + b * FBLK, FBLK)],
                                 wbuf, semw)

            def wait(b, rbuf, wbuf, semg, semw):
                pltpu.make_async_copy(hc.at[d2d.at[b]], rbuf, semg).wait()
                pltpu.make_async_copy(wt_h.at[pl.ds(ebase + b * FBLK, FBLK)],
                                      wbuf, semw).wait()

            def wait_scatter(b, rbuf, sems):
                pltpu.make_async_copy(rbuf, acc.at[s2d.at[b]], sems).wait()

            def phase(b, rbuf, wbuf, semg, semw, sems,
                      rnx, wnx, semgn, semwn, semsn):
                wait(b, rbuf, wbuf, semg, semw)

                @pl.when(b >= 1)
                def _():
                    wait_scatter(b - 1, rnx, semsn)

                @pl.when(b + 1 < FNBLK)
                def _():
                    issue(b + 1, rnx, wnx, semgn, semwn)

                @plsc.parallel_loop(0, FBLK, unroll=4)
                def _(e):
                    wrow = wbuf[e, :]
                    wv = wrow[lanes_c]
                    for j in range(8):
                        rbuf[e, pl.ds(j * 16, 16)] = (
                            rbuf[e, pl.ds(j * 16, 16)] * wv)

                pltpu.async_copy(rbuf, acc.at[s2d.at[b]], sems, add=True)

            issue(0, r0, wb0, sg0, sw0)

            def pair(g, c2):
                phase(2 * g, r0, wb0, sg0, sw0, ss0, r1, wb1, sg1, sw1, ss1)
                phase(2 * g + 1, r1, wb1, sg1, sw1, ss1, r0, wb0, sg0, sw0, ss0)
                return c2

            lax.fori_loop(0, FNBLK // 2, pair, 0)
            wait_scatter(FNBLK - 1, r1, ss1)
            plsc.subcore_barrier()
            lax.fori_loop(0, FBLK, zero_r0, 0)
            for piece in range(ZROWS // FBLK):
                off = tid * ZROWS + piece * FBLK
                pltpu.sync_copy(acc.at[pl.ds(off, FBLK)],
                                out_h.at[c, pl.ds(off, FBLK)])
                pltpu.sync_copy(r0, acc.at[pl.ds(off, FBLK)])
            plsc.subcore_barrier()
            return carry

        lax.fori_loop(0, chc, chunk, 0)

    return k(h3, wt, src2d, dst2d)


def kernel(x, edge_index, w1, a1, w2, a2, bn_gamma, bn_beta):
    f32 = jnp.float32
    x_pad = jnp.pad(x, ((0, NP - N), (0, 0)))
    pad_idx = jnp.full((EP - E,), PAD_NODE, jnp.int32)
    srcp = jnp.concatenate([edge_index[0], pad_idx])
    dstp = jnp.concatenate([edge_index[1], pad_idx])
    src2d = srcp.reshape(EP // BLK, BLK)
    dst2d = dstp.reshape(EP // BLK, BLK)
    src2f = srcp.reshape(EP // FBLK, FBLK)
    dst2f = dstp.reshape(EP // FBLK, FBLK)

    w_all = jnp.transpose(w1, (1, 0, 2)).reshape(F, NH1 * HID)
    amat1 = jnp.zeros((NH1 * HID, 32), f32)
    for h in range(NH1):
        amat1 = amat1.at[h * HID:(h + 1) * HID, h].set(a1[h, :HID])
        amat1 = amat1.at[h * HID:(h + 1) * HID, 16 + h].set(a1[h, HID:])
    amat2 = (jnp.zeros((HID, 32), f32)
             .at[:, 0].set(a2[:HID])
             .at[:, 16].set(a2[HID:]))

    h1 = _mm_call(x_pad, w_all, ncols=16)
    acat1 = _alpha_call(h1, amat1)
    wt1, rs1 = _edgew_call(acat1, src2d, dst2d, nh=NH1)
    hp1 = _featpass_call(h1, wt1, src2f, dst2f, nh=NH1)
    x1 = _combine1_call(rs1, hp1)

    h2 = _mm_call(x1, w2, ncols=2)
    acat2 = _alpha_call(h2, amat2)
    wt2, rs2 = _edgew_call(acat2, src2d, dst2d, nh=1)
    hp2 = _featpass_call(h2, wt2, src2f, dst2f, nh=1)
    out = _final_call(rs2, hp2, bn_gamma, bn_beta)
    return out[:N]
